# baseline (speedup 1.0000x reference)
import functools
import jax
import jax.numpy as jnp
from jax import lax
from jax.experimental import pallas as pl
from jax.experimental.pallas import tpu as pltpu
from jax.experimental.pallas import tpu_sc as plsc

CHARGE_DIM = 64
B, L, D = 16, 4096, 64
NC, NS = 2, 16
CHUNK = 1024


def kernel(sequence, charge, length):
    mesh = plsc.VectorSubcoreMesh(core_axis_name="c", subcore_axis_name="s")

    @functools.partial(
        pl.kernel,
        out_type=jax.ShapeDtypeStruct((B, D, L), jnp.float32),
        mesh=mesh,
        scratch_types=[
            pltpu.VMEM((16,), jnp.float32),
            pltpu.VMEM((16,), jnp.int32),
            pltpu.VMEM((8, 2 * CHUNK), jnp.float32),
            pltpu.VMEM((8, 128), jnp.float32),
            pltpu.SemaphoreType.DMA,
        ],
        compiler_params=pltpu.CompilerParams(
            use_tc_tiling_on_sc=True, needs_layout_passes=False
        ),
    )
    def k(charge_hbm, length_hbm, out_hbm, chv, lnv, buf, btile, sem):
        wid = lax.axis_index("c") * NS + lax.axis_index("s")
        b = wid // 2
        h = wid % 2

        pltpu.sync_copy(charge_hbm, chv)
        pltpu.sync_copy(length_hbm, lnv)

        lane = lax.iota(jnp.int32, 16)
        sel = lane == b
        my_charge = jnp.sum(jnp.where(sel, chv[...], jnp.float32(0.0)))
        my_len = jnp.sum(jnp.where(sel, lnv[...], jnp.int32(0)))

        full = my_len // 128
        rem = my_len - full * 128

        chvec = lax.broadcast(my_charge, (16,))
        zvec = jnp.zeros((16,), jnp.float32)

        def fill(i, _):
            c = i * 16
            for r in range(8):
                buf[r, pl.ds(c, 16)] = chvec
                buf[r, pl.ds(CHUNK + c, 16)] = zvec
            return 0

        lax.fori_loop(0, CHUNK // 16, fill, 0)

        for g in range(8):
            bval = jnp.where(g * 16 + lane < rem, my_charge, jnp.float32(0.0))
            for r in range(8):
                btile[r, pl.ds(g * 16, 16)] = bval

        copies = []
        for j in range(4):
            dt = h * 4 + j
            for c in range(L // CHUNK):
                s = pl.multiple_of(
                    jnp.clip((c + 1) * CHUNK - full * 128, 0, CHUNK), 128
                )
                copies.append(
                    pltpu.async_copy(
                        buf.at[:, pl.ds(s, CHUNK)],
                        out_hbm.at[b, pl.ds(dt * 8, 8), pl.ds(c * CHUNK, CHUNK)],
                        sem,
                    )
                )
        for cp in copies:
            cp.wait()

        @pl.when(full < L // 128)
        def _():
            bcopies = []
            for j in range(4):
                dt = h * 4 + j
                bcopies.append(
                    pltpu.async_copy(
                        btile.at[:, :],
                        out_hbm.at[
                            b, pl.ds(dt * 8, 8),
                            pl.ds(pl.multiple_of(full * 128, 128), 128),
                        ],
                        sem,
                    )
                )
            for cp in bcopies:
                cp.wait()

    out_bdl = k(charge, length)
    return out_bdl.transpose(0, 2, 1)
